# baseline (speedup 1.0000x reference)
import functools

import jax
import jax.numpy as jnp
from jax.experimental import pallas as pl

MASK_RATIO = 0.15
SPAN_MIN = 3
SPAN_MAX = 10


def _maskgen_body(idx_ref, sgn_ref, m_ref, t128_ref, t128s_ref, cu_ref, out_ref,
                  *, rows, n_chunks, lanes, target):
    idx = idx_ref[0]
    hi = idx >> 7
    lo = idx & 127
    sgn = sgn_ref[...]
    iota_c = jax.lax.broadcasted_iota(jnp.int32, (1, 1, n_chunks), 2)
    iota_l = jax.lax.broadcasted_iota(jnp.int32, (1, 1, lanes), 2)
    hc = jnp.where(hi[:, :, None] == iota_c, sgn[:, :, None], 0.0).astype(jnp.bfloat16)
    hl = (lo[:, :, None] == iota_l).astype(jnp.bfloat16)
    delta = jax.lax.dot_general(
        hc, hl, (((1,), (1,)), ((0,), (0,))),
        preferred_element_type=jnp.float32)
    t128 = t128_ref[...]
    within = jax.lax.dot_general(
        delta, t128, (((2,), (0,)), ((), ())),
        preferred_element_type=jnp.float32)
    rowtot = jnp.sum(delta, axis=2)
    cu = cu_ref[...]
    carry = jax.lax.dot_general(
        rowtot, cu, (((1,), (0,)), ((), ())),
        preferred_element_type=jnp.float32)
    painted = (within + carry[:, :, None]) > 0.5

    m = m_ref[...].reshape(rows, n_chunks, lanes)
    key = m + jnp.where(painted, 1 << 23, 0)

    t = jnp.zeros((rows, 1, 1), jnp.int32)
    for b in range(23, -1, -1):
        cand = t + (1 << b)
        cnt = jnp.sum((key >= cand).astype(jnp.float32), axis=(1, 2),
                      keepdims=True)
        t = jnp.where(cnt >= float(target), cand, t)
    gt = key > t
    cnt_gt = jnp.sum(gt.astype(jnp.float32), axis=(1, 2), keepdims=True)
    deficit = float(target) - cnt_gt
    eq = (key == t).astype(jnp.float32)
    t128s = t128s_ref[...]
    eq_within = jax.lax.dot_general(
        eq, t128s, (((2,), (0,)), ((), ())),
        preferred_element_type=jnp.float32)
    eq_tot = jnp.sum(eq, axis=2)
    eq_carry = jax.lax.dot_general(
        eq_tot, cu, (((1,), (0,)), ((), ())),
        preferred_element_type=jnp.float32)
    eq_rank = eq_within + eq_carry[:, :, None]
    adjusted = gt | ((eq > 0.5) & (eq_rank < deficit))
    out_ref[...] = adjusted.astype(jnp.bfloat16).reshape(rows * n_chunks, lanes)


def kernel(features, mask_token):
    batch, seq_len, n_features = features.shape
    target = int(seq_len * MASK_RATIO)
    avg_span = (SPAN_MIN + SPAN_MAX) / 2.0
    n_spans = max(1, int(target / avg_span * 2))

    k1, k2, k3 = jax.random.split(jax.random.key(42), 3)
    span_lengths = jax.random.randint(k1, (batch, n_spans), SPAN_MIN, SPAN_MAX + 1)
    starts = jax.random.randint(k2, (batch, n_spans), 0, max(1, seq_len - SPAN_MIN))
    ends = jnp.minimum(starts + span_lengths, seq_len)
    idx = jnp.concatenate([starts, ends], axis=1).astype(jnp.int32)
    rows = 32
    idx = idx.reshape(batch // rows, rows, 2 * n_spans)
    sgn = jnp.concatenate([jnp.ones((1, n_spans), jnp.float32),
                           -jnp.ones((1, n_spans), jnp.float32)], axis=1)
    u = jax.random.uniform(k3, (batch, seq_len), dtype=jnp.float32)
    m23 = (u * float(1 << 23)).astype(jnp.int32).reshape(batch * seq_len // 128, 128)

    lanes = 128
    n_chunks = seq_len // lanes
    grid_mg = (batch // rows,)

    a = jax.lax.broadcasted_iota(jnp.int32, (lanes, lanes), 0)
    b = jax.lax.broadcasted_iota(jnp.int32, (lanes, lanes), 1)
    t128 = (a <= b).astype(jnp.float32)
    t128s = (a < b).astype(jnp.float32)
    ac = jax.lax.broadcasted_iota(jnp.int32, (n_chunks, n_chunks), 0)
    bc = jax.lax.broadcasted_iota(jnp.int32, (n_chunks, n_chunks), 1)
    cu = (ac < bc).astype(jnp.float32)

    masks_bf = pl.pallas_call(
        functools.partial(_maskgen_body, rows=rows, n_chunks=n_chunks,
                          lanes=lanes, target=target),
        grid=grid_mg,
        in_specs=[
            pl.BlockSpec((1, rows, 2 * n_spans), lambda i: (i, 0, 0)),
            pl.BlockSpec((1, 2 * n_spans), lambda i: (0, 0)),
            pl.BlockSpec((rows * n_chunks, lanes), lambda i: (i, 0)),
            pl.BlockSpec((lanes, lanes), lambda i: (0, 0)),
            pl.BlockSpec((lanes, lanes), lambda i: (0, 0)),
            pl.BlockSpec((n_chunks, n_chunks), lambda i: (0, 0)),
        ],
        out_specs=pl.BlockSpec((rows * n_chunks, lanes), lambda i: (i, 0)),
        out_shape=jax.ShapeDtypeStruct((batch * n_chunks, lanes), jnp.bfloat16),
    )(idx, sgn, m23, t128, t128s, cu)

    masks = masks_bf.reshape(batch, seq_len) > 0.5
    masks = masks & ~jnp.isnan(features[:, :, 0])
    return jnp.where(masks[:, :, None], mask_token[None, None, :], features)


# speedup vs baseline: 1.2206x; 1.2206x over previous
import functools

import jax
import jax.numpy as jnp
from jax.experimental import pallas as pl

MASK_RATIO = 0.15
SPAN_MIN = 3
SPAN_MAX = 10


def _maskgen_body(idx_ref, sgn_ref, m_ref, t128_ref, t128s_ref, cu_ref, out_ref,
                  *, rows, n_chunks, lanes, target):
    idx = idx_ref[0]
    hi = idx >> 7
    lo = idx & 127
    sgn = sgn_ref[...]
    iota_c = jax.lax.broadcasted_iota(jnp.int32, (1, n_chunks, 1), 1)
    iota_l = jax.lax.broadcasted_iota(jnp.int32, (1, 1, lanes), 2)
    hc = jnp.where(hi[:, None, :] == iota_c, sgn[:, None, :], 0.0).astype(jnp.bfloat16)
    hl = (lo[:, :, None] == iota_l).astype(jnp.bfloat16)
    delta = jax.lax.dot_general(
        hc, hl, (((2,), (1,)), ((0,), (0,))),
        preferred_element_type=jnp.float32)
    t128 = t128_ref[...]
    within = jax.lax.dot_general(
        delta, t128, (((2,), (0,)), ((), ())),
        preferred_element_type=jnp.float32)
    rowtot = jnp.sum(delta, axis=2)
    cu = cu_ref[...]
    carry = jax.lax.dot_general(
        rowtot, cu, (((1,), (0,)), ((), ())),
        preferred_element_type=jnp.float32)
    painted = (within + carry[:, :, None]) > 0.5

    m = m_ref[...].reshape(rows, n_chunks, lanes)
    key = m + jnp.where(painted, 1 << 23, 0)

    t = jnp.zeros((rows, 1, 1), jnp.int32)
    for b in range(23, -1, -1):
        cand = t + (1 << b)
        s1 = jnp.sum((key >= cand).astype(jnp.float32), axis=1)
        cnt = jnp.sum(s1, axis=1)[:, None, None]
        t = jnp.where(cnt >= float(target), cand, t)
    gt = key > t
    cnt_gt = jnp.sum(gt.astype(jnp.float32), axis=(1, 2), keepdims=True)
    deficit = float(target) - cnt_gt
    eq = (key == t).astype(jnp.float32)
    t128s = t128s_ref[...]
    eq_within = jax.lax.dot_general(
        eq, t128s, (((2,), (0,)), ((), ())),
        preferred_element_type=jnp.float32)
    eq_tot = jnp.sum(eq, axis=2)
    eq_carry = jax.lax.dot_general(
        eq_tot, cu, (((1,), (0,)), ((), ())),
        preferred_element_type=jnp.float32)
    eq_rank = eq_within + eq_carry[:, :, None]
    adjusted = gt | ((eq > 0.5) & (eq_rank < deficit))
    out_ref[...] = adjusted.astype(jnp.bfloat16).reshape(rows * n_chunks, lanes)


def kernel(features, mask_token):
    batch, seq_len, n_features = features.shape
    target = int(seq_len * MASK_RATIO)
    avg_span = (SPAN_MIN + SPAN_MAX) / 2.0
    n_spans = max(1, int(target / avg_span * 2))

    k1, k2, k3 = jax.random.split(jax.random.key(42), 3)
    span_lengths = jax.random.randint(k1, (batch, n_spans), SPAN_MIN, SPAN_MAX + 1)
    starts = jax.random.randint(k2, (batch, n_spans), 0, max(1, seq_len - SPAN_MIN))
    ends = jnp.minimum(starts + span_lengths, seq_len)
    idx = jnp.concatenate([starts, ends], axis=1).astype(jnp.int32)
    rows = 64
    idx = idx.reshape(batch // rows, rows, 2 * n_spans)
    sgn = jnp.concatenate([jnp.ones((1, n_spans), jnp.float32),
                           -jnp.ones((1, n_spans), jnp.float32)], axis=1)
    u = jax.random.uniform(k3, (batch, seq_len), dtype=jnp.float32)
    m23 = (u * float(1 << 23)).astype(jnp.int32).reshape(batch * seq_len // 128, 128)

    lanes = 128
    n_chunks = seq_len // lanes
    grid_mg = (batch // rows,)

    a = jax.lax.broadcasted_iota(jnp.int32, (lanes, lanes), 0)
    b = jax.lax.broadcasted_iota(jnp.int32, (lanes, lanes), 1)
    t128 = (a <= b).astype(jnp.float32)
    t128s = (a < b).astype(jnp.float32)
    ac = jax.lax.broadcasted_iota(jnp.int32, (n_chunks, n_chunks), 0)
    bc = jax.lax.broadcasted_iota(jnp.int32, (n_chunks, n_chunks), 1)
    cu = (ac < bc).astype(jnp.float32)

    masks_bf = pl.pallas_call(
        functools.partial(_maskgen_body, rows=rows, n_chunks=n_chunks,
                          lanes=lanes, target=target),
        grid=grid_mg,
        in_specs=[
            pl.BlockSpec((1, rows, 2 * n_spans), lambda i: (i, 0, 0)),
            pl.BlockSpec((1, 2 * n_spans), lambda i: (0, 0)),
            pl.BlockSpec((rows * n_chunks, lanes), lambda i: (i, 0)),
            pl.BlockSpec((lanes, lanes), lambda i: (0, 0)),
            pl.BlockSpec((lanes, lanes), lambda i: (0, 0)),
            pl.BlockSpec((n_chunks, n_chunks), lambda i: (0, 0)),
        ],
        out_specs=pl.BlockSpec((rows * n_chunks, lanes), lambda i: (i, 0)),
        out_shape=jax.ShapeDtypeStruct((batch * n_chunks, lanes), jnp.bfloat16),
    )(idx, sgn, m23, t128, t128s, cu)

    masks = masks_bf.reshape(batch, seq_len) > 0.5
    masks = masks & ~jnp.isnan(features[:, :, 0])
    return jnp.where(masks[:, :, None], mask_token[None, None, :], features)


# speedup vs baseline: 1.2297x; 1.0075x over previous
import functools

import jax
import jax.numpy as jnp
from jax.experimental import pallas as pl

MASK_RATIO = 0.15
SPAN_MIN = 3
SPAN_MAX = 10


def _maskgen_body(idx_ref, sgn_ref, m_ref, t128_ref, t128s_ref, cu_ref, out_ref,
                  *, rows, n_chunks, lanes, target):
    idx = idx_ref[0]
    hi = idx >> 7
    lo = idx & 127
    sgn = sgn_ref[...]
    iota_c = jax.lax.broadcasted_iota(jnp.int32, (1, n_chunks, 1), 1)
    iota_l = jax.lax.broadcasted_iota(jnp.int32, (1, 1, lanes), 2)
    hc = jnp.where(hi[:, None, :] == iota_c, sgn[:, None, :], 0.0).astype(jnp.bfloat16)
    hl = (lo[:, :, None] == iota_l).astype(jnp.bfloat16)
    delta = jax.lax.dot_general(
        hc, hl, (((2,), (1,)), ((0,), (0,))),
        preferred_element_type=jnp.float32)
    t128 = t128_ref[...]
    within = jax.lax.dot_general(
        delta, t128, (((2,), (0,)), ((), ())),
        preferred_element_type=jnp.float32)
    rowtot = jnp.sum(delta, axis=2)
    cu = cu_ref[...]
    carry = jax.lax.dot_general(
        rowtot, cu, (((1,), (0,)), ((), ())),
        preferred_element_type=jnp.float32)
    painted = (within + carry[:, :, None]) > 0.5

    m = m_ref[...].reshape(rows, n_chunks, lanes)
    key = m + jnp.where(painted, 1 << 23, 0)

    t = jnp.zeros((rows, 1, 1), jnp.int32)
    for b in range(23, -1, -1):
        cand = t + (1 << b)
        s1 = jnp.sum((key >= cand).astype(jnp.float32), axis=1)
        cnt = jnp.sum(s1, axis=1)[:, None, None]
        t = jnp.where(cnt >= float(target), cand, t)
    gt = key > t
    cnt_gt = jnp.sum(gt.astype(jnp.float32), axis=(1, 2), keepdims=True)
    deficit = float(target) - cnt_gt
    eq = (key == t).astype(jnp.float32)
    t128s = t128s_ref[...]
    eq_within = jax.lax.dot_general(
        eq, t128s, (((2,), (0,)), ((), ())),
        preferred_element_type=jnp.float32)
    eq_tot = jnp.sum(eq, axis=2)
    eq_carry = jax.lax.dot_general(
        eq_tot, cu, (((1,), (0,)), ((), ())),
        preferred_element_type=jnp.float32)
    eq_rank = eq_within + eq_carry[:, :, None]
    adjusted = gt | ((eq > 0.5) & (eq_rank < deficit))
    out_ref[...] = adjusted.astype(jnp.bfloat16).reshape(rows * n_chunks, lanes)


def kernel(features, mask_token):
    batch, seq_len, n_features = features.shape
    target = int(seq_len * MASK_RATIO)
    avg_span = (SPAN_MIN + SPAN_MAX) / 2.0
    n_spans = max(1, int(target / avg_span * 2))

    k1, k2, k3 = jax.random.split(jax.random.key(42), 3)
    span_lengths = jax.random.randint(k1, (batch, n_spans), SPAN_MIN, SPAN_MAX + 1)
    starts = jax.random.randint(k2, (batch, n_spans), 0, max(1, seq_len - SPAN_MIN))
    ends = jnp.minimum(starts + span_lengths, seq_len)
    idx = jnp.concatenate([starts, ends], axis=1).astype(jnp.int32)
    rows = 64
    idx = idx.reshape(batch // rows, rows, 2 * n_spans)
    sgn = jnp.concatenate([jnp.ones((1, n_spans), jnp.float32),
                           -jnp.ones((1, n_spans), jnp.float32)], axis=1)
    m23 = (jax.random.bits(k3, (batch, seq_len), jnp.uint32) >> 9
           ).astype(jnp.int32).reshape(batch * seq_len // 128, 128)

    lanes = 128
    n_chunks = seq_len // lanes
    grid_mg = (batch // rows,)

    a = jax.lax.broadcasted_iota(jnp.int32, (lanes, lanes), 0)
    b = jax.lax.broadcasted_iota(jnp.int32, (lanes, lanes), 1)
    t128 = (a <= b).astype(jnp.float32)
    t128s = (a < b).astype(jnp.float32)
    ac = jax.lax.broadcasted_iota(jnp.int32, (n_chunks, n_chunks), 0)
    bc = jax.lax.broadcasted_iota(jnp.int32, (n_chunks, n_chunks), 1)
    cu = (ac < bc).astype(jnp.float32)

    masks_bf = pl.pallas_call(
        functools.partial(_maskgen_body, rows=rows, n_chunks=n_chunks,
                          lanes=lanes, target=target),
        grid=grid_mg,
        in_specs=[
            pl.BlockSpec((1, rows, 2 * n_spans), lambda i: (i, 0, 0)),
            pl.BlockSpec((1, 2 * n_spans), lambda i: (0, 0)),
            pl.BlockSpec((rows * n_chunks, lanes), lambda i: (i, 0)),
            pl.BlockSpec((lanes, lanes), lambda i: (0, 0)),
            pl.BlockSpec((lanes, lanes), lambda i: (0, 0)),
            pl.BlockSpec((n_chunks, n_chunks), lambda i: (0, 0)),
        ],
        out_specs=pl.BlockSpec((rows * n_chunks, lanes), lambda i: (i, 0)),
        out_shape=jax.ShapeDtypeStruct((batch * n_chunks, lanes), jnp.bfloat16),
    )(idx, sgn, m23, t128, t128s, cu)

    masks = masks_bf.reshape(batch, seq_len) > 0.5
    masks = masks & ~jnp.isnan(features[:, :, 0])
    return jnp.where(masks[:, :, None], mask_token[None, None, :], features)
